# Initial kernel scaffold; baseline (speedup 1.0000x reference)
import jax
import jax.numpy as jnp
from jax.experimental import pallas as pl

B = 4
F = 128
H = 256
W = 256
CB = 32


def _pos_kernel(colT_ref, rowT_ref, out_ref):
    half = pl.program_id(1)

    @pl.when(half == 0)
    def _():
        out_ref[0] = jnp.broadcast_to(colT_ref[:, None, :], (CB, H, W))

    @pl.when(half == 1)
    def _():
        out_ref[0] = jnp.broadcast_to(rowT_ref[:, :, None], (CB, H, W))


def kernel(bev_mask, row_embed, col_embed):
    colT = col_embed.T
    rowT = row_embed.T

    grid = (B, 2, F // CB)
    out = pl.pallas_call(
        _pos_kernel,
        grid=grid,
        in_specs=[
            pl.BlockSpec((CB, W), lambda b, s, j: (j, 0)),
            pl.BlockSpec((CB, H), lambda b, s, j: (j, 0)),
        ],
        out_specs=pl.BlockSpec(
            (1, CB, H, W), lambda b, s, j: (b, s * (F // CB) + j, 0, 0)
        ),
        out_shape=jax.ShapeDtypeStruct((B, 2 * F, H, W), jnp.float32),
    )(colT, rowT)
    return out


# speedup vs baseline: 2.0280x; 2.0280x over previous
import jax
import jax.numpy as jnp
from jax.experimental import pallas as pl

B = 4
F = 128
H = 256
W = 256
CB = 32


def _pos_kernel(colT_ref, rowT_ref, out_ref):
    half = pl.program_id(1)

    @pl.when(half == 0)
    def _():
        out_ref[0] = jnp.broadcast_to(colT_ref[...][:, None, :], (CB, H, W))

    @pl.when(half == 1)
    def _():
        out_ref[0] = jnp.broadcast_to(rowT_ref[...][:, :, None], (CB, H, W))


def kernel(bev_mask, row_embed, col_embed):
    colT = col_embed.T
    rowT = row_embed.T

    grid = (B, 2, F // CB)
    out = pl.pallas_call(
        _pos_kernel,
        grid=grid,
        in_specs=[
            pl.BlockSpec((CB, W), lambda b, s, j: (j, 0)),
            pl.BlockSpec((CB, H), lambda b, s, j: (j, 0)),
        ],
        out_specs=pl.BlockSpec(
            (1, CB, H, W), lambda b, s, j: (b, s * (F // CB) + j, 0, 0)
        ),
        out_shape=jax.ShapeDtypeStruct((B, 2 * F, H, W), jnp.float32),
    )(colT, rowT)
    return out
